# Initial kernel scaffold; baseline (speedup 1.0000x reference)
import jax
import jax.numpy as jnp
from jax.experimental import pallas as pl


def kernel(x_nchw, conv_w, conv_b, bn_gamma, bn_beta):
    raise NotImplementedError("write your pallas kernel here")


# speedup vs baseline: 1.3877x; 1.3877x over previous
import functools

import jax
import jax.numpy as jnp
from jax.experimental import pallas as pl
from jax.experimental.pallas import tpu as pltpu

_EPS = 1e-5


def _stats_compact_kernel(x_hbm, sel_ref, x2_ref, gram_ref, xbuf, sems, *, nb,
                          cin, ho, w, wo, j_steps):
    c = pl.program_id(0)
    j = pl.program_id(1)
    chunk = c * j_steps + j
    slot = jax.lax.rem(j, 2)
    nslot = jax.lax.rem(j + 1, 2)

    def _start(ch, sl):
        pltpu.make_async_copy(
            x_hbm.at[pl.ds(ch * nb, nb), :, :, 0, :],
            xbuf.at[sl], sems.at[sl]).start()

    @pl.when(j == 0)
    def _():
        _start(chunk, slot)

    @pl.when(j + 1 < j_steps)
    def _():
        _start(chunk + 1, nslot)

    pltpu.make_async_copy(
        x_hbm.at[pl.ds(0, nb), :, :, 0, :], xbuf.at[slot],
        sems.at[slot]).wait()

    pairs = [(i, k) for i in range(cin) for k in range(i, cin)]
    acc = [None] * (cin + len(pairs))
    for b in range(nb):
        vb = xbuf[slot, b]
        vflat = vb.reshape(cin * ho, w).astype(jnp.bfloat16)
        xc = jnp.dot(vflat, sel_ref[...],
                     preferred_element_type=jnp.float32)
        x2_ref[b] = xc.reshape(cin, ho, wo).astype(jnp.bfloat16)
        ch = [xc[i * ho:(i + 1) * ho] for i in range(cin)]
        parts = [ch[i] for i in range(cin)]
        parts += [ch[i] * ch[k] for (i, k) in pairs]
        for r, t in enumerate(parts):
            s = jnp.sum(t, axis=0, keepdims=True)
            acc[r] = s if acc[r] is None else acc[r] + s

    rows = gram_ref.shape[0]
    row_iota = jax.lax.broadcasted_iota(jnp.int32, (rows, wo), 0)
    z = jnp.zeros((rows, wo), jnp.float32)
    for r, a in enumerate(acc):
        z = jnp.where(row_iota == r, a, z)
    gram_ref[...] = z


def _conv_bn_relu_kernel(x_ref, wblk_ref, shift_ref, o_ref):
    y = jnp.dot(wblk_ref[...], x_ref[...], preferred_element_type=jnp.float32)
    o_ref[...] = jnp.maximum(y + shift_ref[...], 0.0)


@jax.jit
def kernel(x_nchw, conv_w, conv_b, bn_gamma, bn_beta):
    n, cin, h, w = x_nchw.shape
    cout = conv_w.shape[0]
    ho, wo = (h + 1) // 2, (w + 1) // 2
    p = ho * wo
    del conv_b
    w2 = conv_w.reshape(cout, cin).astype(jnp.float32)

    nb = next(d for d in (8, 4, 2, 1) if n % (2 * d) == 0)
    g1 = n // nb
    j_steps = g1 // 2
    stat_rows = cin + (cin * (cin + 1)) // 2

    x5 = x_nchw.reshape(n, cin, ho, 2, w)
    sel = (jax.lax.broadcasted_iota(jnp.int32, (w, wo), 0) ==
           2 * jax.lax.broadcasted_iota(jnp.int32, (w, wo), 1)
           ).astype(jnp.bfloat16)
    x2c, gram = pl.pallas_call(
        functools.partial(_stats_compact_kernel, nb=nb, cin=cin, ho=ho, w=w,
                          wo=wo, j_steps=j_steps),
        out_shape=(jax.ShapeDtypeStruct((n, cin, ho, wo), jnp.bfloat16),
                   jax.ShapeDtypeStruct((g1, stat_rows, wo), jnp.float32)),
        grid=(2, j_steps),
        in_specs=[pl.BlockSpec(memory_space=pl.ANY),
                  pl.BlockSpec((w, wo), lambda c, j: (0, 0))],
        out_specs=(
            pl.BlockSpec((nb, cin, ho, wo),
                         lambda c, j: (c * j_steps + j, 0, 0, 0)),
            pl.BlockSpec((None, stat_rows, wo),
                         lambda c, j: (c * j_steps + j, 0, 0)),
        ),
        scratch_shapes=[
            pltpu.VMEM((2, nb, cin, ho, w), jnp.float32),
            pltpu.SemaphoreType.DMA((2,)),
        ],
        compiler_params=pltpu.CompilerParams(
            dimension_semantics=("parallel", "arbitrary")),
        name="stats_compact",
    )(x5, sel)

    g = jnp.sum(gram, axis=(0, 2))
    s = g[:cin]
    iu = jnp.triu_indices(cin)
    m_up = jnp.zeros((cin, cin), jnp.float32).at[iu].set(g[cin:])
    m_full = m_up + m_up.T - jnp.diag(jnp.diag(m_up))
    inv_count = 1.0 / float(n * p)
    mean_y = (w2 @ s) * inv_count
    ey2 = jnp.einsum("oc,cd,od->o", w2, m_full, w2) * inv_count
    var = jnp.maximum(ey2 - mean_y * mean_y, 0.0)
    scale = bn_gamma * jax.lax.rsqrt(var + _EPS)
    shift = bn_beta - mean_y * scale
    wf = scale[:, None] * w2

    nb2 = next(d for d in (8, 4, 2, 1) if n % (2 * d) == 0)
    g2 = n // nb2
    wblk = jnp.kron(jnp.eye(nb2, dtype=jnp.float32), wf).astype(jnp.bfloat16)
    shift_blk = jnp.tile(shift[:, None], (nb2, 1))
    x2r = x2c.reshape(n * cin, p)

    out_flat = pl.pallas_call(
        _conv_bn_relu_kernel,
        out_shape=jax.ShapeDtypeStruct((n * cout, p), jnp.float32),
        grid=(g2,),
        in_specs=[
            pl.BlockSpec((nb2 * cin, p), lambda i: (i, 0)),
            pl.BlockSpec((nb2 * cout, nb2 * cin), lambda i: (0, 0)),
            pl.BlockSpec((nb2 * cout, 1), lambda i: (0, 0)),
        ],
        out_specs=pl.BlockSpec((nb2 * cout, p), lambda i: (i, 0)),
        compiler_params=pltpu.CompilerParams(
            dimension_semantics=("parallel",)),
        name="folded_conv_bn_relu",
    )(x2r, wblk, shift_blk)

    return out_flat.reshape(n, cout, ho, wo)


# speedup vs baseline: 1.5805x; 1.1389x over previous
import functools

import jax
import jax.numpy as jnp
from jax.experimental import pallas as pl
from jax.experimental.pallas import tpu as pltpu

_EPS = 1e-5


def _stats_compact_kernel(x_ref, selw_ref, selh_ref, x2_ref, gram_ref, *, nb,
                          cin, ho, wo):
    pairs = [(i, k) for i in range(cin) for k in range(i, cin)]
    acc = [None] * (cin + len(pairs))
    for b in range(nb):
        vb = x_ref[b].astype(jnp.bfloat16)
        t = jnp.dot(vb, selw_ref[...],
                    preferred_element_type=jnp.float32
                    ).astype(jnp.bfloat16)
        xc = jnp.dot(selh_ref[...], t,
                     preferred_element_type=jnp.float32)
        x2_ref[b] = xc.reshape(cin, ho, wo).astype(jnp.bfloat16)
        ch = [xc[i * ho:(i + 1) * ho] for i in range(cin)]
        parts = [ch[i] for i in range(cin)]
        parts += [ch[i] * ch[k] for (i, k) in pairs]
        for r, t2 in enumerate(parts):
            s = jnp.sum(t2, axis=0, keepdims=True)
            acc[r] = s if acc[r] is None else acc[r] + s

    rows = gram_ref.shape[0]
    row_iota = jax.lax.broadcasted_iota(jnp.int32, (rows, wo), 0)
    z = jnp.zeros((rows, wo), jnp.float32)
    for r, a in enumerate(acc):
        z = jnp.where(row_iota == r, a, z)
    gram_ref[...] = z


def _conv_bn_relu_kernel(x_ref, wblk_ref, shift_ref, o_ref):
    y = jnp.dot(wblk_ref[...], x_ref[...], preferred_element_type=jnp.float32)
    o_ref[...] = jnp.maximum(y + shift_ref[...], 0.0)


@jax.jit
def kernel(x_nchw, conv_w, conv_b, bn_gamma, bn_beta):
    n, cin, h, w = x_nchw.shape
    cout = conv_w.shape[0]
    ho, wo = (h + 1) // 2, (w + 1) // 2
    p = ho * wo
    del conv_b
    w2 = conv_w.reshape(cout, cin).astype(jnp.float32)

    nb = next(d for d in (8, 4, 2, 1) if n % (2 * d) == 0)
    g1 = n // nb
    stat_rows = cin + (cin * (cin + 1)) // 2

    selw = (jax.lax.broadcasted_iota(jnp.int32, (w, wo), 0) ==
            2 * jax.lax.broadcasted_iota(jnp.int32, (w, wo), 1)
            ).astype(jnp.bfloat16)
    selh = (jax.lax.broadcasted_iota(jnp.int32, (ho, h), 1) ==
            2 * jax.lax.broadcasted_iota(jnp.int32, (ho, h), 0)
            ).astype(jnp.bfloat16)
    selh_ch = jnp.kron(jnp.eye(cin, dtype=jnp.bfloat16), selh)

    x3 = x_nchw.reshape(n, cin * h, w)
    x2c, gram = pl.pallas_call(
        functools.partial(_stats_compact_kernel, nb=nb, cin=cin, ho=ho,
                          wo=wo),
        out_shape=(jax.ShapeDtypeStruct((n, cin, ho, wo), jnp.bfloat16),
                   jax.ShapeDtypeStruct((g1, stat_rows, wo), jnp.float32)),
        grid=(g1,),
        in_specs=[pl.BlockSpec((nb, cin * h, w), lambda i: (i, 0, 0)),
                  pl.BlockSpec((w, wo), lambda i: (0, 0)),
                  pl.BlockSpec((cin * ho, cin * h), lambda i: (0, 0))],
        out_specs=(
            pl.BlockSpec((nb, cin, ho, wo), lambda i: (i, 0, 0, 0)),
            pl.BlockSpec((None, stat_rows, wo), lambda i: (i, 0, 0)),
        ),
        compiler_params=pltpu.CompilerParams(
            dimension_semantics=("parallel",)),
        name="stats_compact",
    )(x3, selw, selh_ch)

    g = jnp.sum(gram, axis=(0, 2))
    s = g[:cin]
    iu = jnp.triu_indices(cin)
    m_up = jnp.zeros((cin, cin), jnp.float32).at[iu].set(g[cin:])
    m_full = m_up + m_up.T - jnp.diag(jnp.diag(m_up))
    inv_count = 1.0 / float(n * p)
    mean_y = (w2 @ s) * inv_count
    ey2 = jnp.einsum("oc,cd,od->o", w2, m_full, w2) * inv_count
    var = jnp.maximum(ey2 - mean_y * mean_y, 0.0)
    scale = bn_gamma * jax.lax.rsqrt(var + _EPS)
    shift = bn_beta - mean_y * scale
    wf = scale[:, None] * w2

    nb2 = next(d for d in (8, 4, 2, 1) if n % (2 * d) == 0)
    g2 = n // nb2
    wblk = jnp.kron(jnp.eye(nb2, dtype=jnp.float32), wf).astype(jnp.bfloat16)
    shift_blk = jnp.tile(shift[:, None], (nb2, 1))
    x2r = x2c.reshape(n * cin, p)

    out_flat = pl.pallas_call(
        _conv_bn_relu_kernel,
        out_shape=jax.ShapeDtypeStruct((n * cout, p), jnp.float32),
        grid=(g2,),
        in_specs=[
            pl.BlockSpec((nb2 * cin, p), lambda i: (i, 0)),
            pl.BlockSpec((nb2 * cout, nb2 * cin), lambda i: (0, 0)),
            pl.BlockSpec((nb2 * cout, 1), lambda i: (0, 0)),
        ],
        out_specs=pl.BlockSpec((nb2 * cout, p), lambda i: (i, 0)),
        compiler_params=pltpu.CompilerParams(
            dimension_semantics=("parallel",)),
        name="folded_conv_bn_relu",
    )(x2r, wblk, shift_blk)

    return out_flat.reshape(n, cout, ho, wo)
